# baseline (device time: 25412 ns/iter reference)
import jax
import jax.numpy as jnp
from jax import lax
from jax.experimental import pallas as pl
from jax.experimental.pallas import tpu as pltpu

N_DEV = 32
N_BLOCKS = 4

_NEAR_FIRST_OFFSETS = [
    o for d in range(1, N_DEV // 2 + 1)
    for o in ({d, N_DEV - d} if d != N_DEV - d else {d})
]


def kernel(A, B):
    m, k = A.shape
    k2, n = B.shape
    assert k == k2
    mc = m // N_DEV
    mb = m // N_BLOCKS
    cpb = N_DEV // N_BLOCKS

    def body(a_ref, b_ref, out_ref, partial, rs_buf, red_buf, ag_buf,
             ag_recv, s1, r1, s2, r2):
        my_pos = lax.axis_index("i")

        barrier_sem = pltpu.get_barrier_semaphore()
        pl.semaphore_signal(barrier_sem, inc=1)
        pl.semaphore_wait(barrier_sem, 1)

        def mm_block(g):
            partial[pl.ds(g * mb, mb), :] = jnp.dot(
                a_ref[pl.ds(g * mb, mb), :], b_ref[...],
                preferred_element_type=jnp.float32,
            ).astype(jnp.bfloat16)

        far_block = lax.rem(my_pos // cpb + N_BLOCKS // 2, N_BLOCKS)
        rs_sends = {}
        for j in range(N_BLOCKS):
            g = lax.rem(far_block + j, N_BLOCKS)
            mm_block(g)
            for t in range(cpb):
                c = g * cpb + t
                rdma = pltpu.make_async_remote_copy(
                    src_ref=partial.at[pl.ds(c * mc, mc), :],
                    dst_ref=rs_buf.at[my_pos],
                    send_sem=s1.at[c],
                    recv_sem=r1.at[my_pos],
                    device_id=(c,),
                    device_id_type=pl.DeviceIdType.MESH,
                )
                rs_sends[(j, t)] = (c, rdma)

                @pl.when(c != my_pos)
                def _(rdma=rdma):
                    rdma.start()

        red_buf[...] = partial[pl.ds(my_pos * mc, mc), :].astype(jnp.float32)

        for o in _NEAR_FIRST_OFFSETS:
            q = lax.rem(my_pos - o + N_DEV, N_DEV)
            recv = pltpu.make_async_remote_copy(
                src_ref=rs_buf.at[q],
                dst_ref=rs_buf.at[q],
                send_sem=s1.at[q],
                recv_sem=r1.at[q],
                device_id=(q,),
                device_id_type=pl.DeviceIdType.MESH,
            )
            recv.wait_recv()
            red_buf[...] += rs_buf[q].astype(jnp.float32)

        out_ref[pl.ds(my_pos * mc, mc), :] = red_buf[...]
        ag_buf[...] = red_buf[...].astype(jnp.bfloat16)

        ag_sends = []
        for o in reversed(_NEAR_FIRST_OFFSETS):
            d = lax.rem(my_pos + o, N_DEV)
            rdma = pltpu.make_async_remote_copy(
                src_ref=ag_buf,
                dst_ref=ag_recv.at[my_pos],
                send_sem=s2.at[d],
                recv_sem=r2.at[my_pos],
                device_id=(d,),
                device_id_type=pl.DeviceIdType.MESH,
            )
            ag_sends.append(rdma)
            rdma.start()

        for o in _NEAR_FIRST_OFFSETS:
            q = lax.rem(my_pos - o + N_DEV, N_DEV)
            recv = pltpu.make_async_remote_copy(
                src_ref=ag_recv.at[q],
                dst_ref=ag_recv.at[q],
                send_sem=s2.at[q],
                recv_sem=r2.at[q],
                device_id=(q,),
                device_id_type=pl.DeviceIdType.MESH,
            )
            recv.wait_recv()
            out_ref[pl.ds(q * mc, mc), :] = ag_recv[q].astype(jnp.float32)

        for c, rdma in rs_sends.values():
            @pl.when(c != my_pos)
            def _(rdma=rdma):
                rdma.wait_send()
        for rdma in ag_sends:
            rdma.wait_send()

    return pl.pallas_call(
        body,
        out_shape=jax.ShapeDtypeStruct((m, n), jnp.float32),
        in_specs=[
            pl.BlockSpec(memory_space=pltpu.VMEM),
            pl.BlockSpec(memory_space=pltpu.VMEM),
        ],
        out_specs=pl.BlockSpec(memory_space=pltpu.VMEM),
        scratch_shapes=[
            pltpu.VMEM((m, n), jnp.bfloat16),
            pltpu.VMEM((N_DEV, mc, n), jnp.bfloat16),
            pltpu.VMEM((mc, n), jnp.float32),
            pltpu.VMEM((mc, n), jnp.bfloat16),
            pltpu.VMEM((N_DEV, mc, n), jnp.bfloat16),
            pltpu.SemaphoreType.DMA((N_DEV,)),
            pltpu.SemaphoreType.DMA((N_DEV,)),
            pltpu.SemaphoreType.DMA((N_DEV,)),
            pltpu.SemaphoreType.DMA((N_DEV,)),
        ],
        compiler_params=pltpu.CompilerParams(collective_id=0),
    )(A, B)


# device time: 22869 ns/iter; 1.1112x vs baseline; 1.1112x over previous
import jax
import jax.numpy as jnp
from jax import lax
from jax.experimental import pallas as pl
from jax.experimental.pallas import tpu as pltpu

N_DEV = 32
N_BLOCKS = 4


def kernel(A, B):
    m, k = A.shape
    k2, n = B.shape
    assert k == k2
    mc = m // N_DEV
    mb = m // N_BLOCKS
    cpb = N_DEV // N_BLOCKS

    def body(a_ref, b_ref, out_ref, partial, rs_buf, red_buf, ag_buf,
             ag_recv, s1, r1, s2, r2):
        my_pos = lax.axis_index("i")

        barrier_sem = pltpu.get_barrier_semaphore()
        pl.semaphore_signal(barrier_sem, inc=1)
        pl.semaphore_wait(barrier_sem, 1)

        def mm_block(g):
            partial[pl.ds(g * mb, mb), :] = jnp.dot(
                a_ref[pl.ds(g * mb, mb), :], b_ref[...],
                preferred_element_type=jnp.float32,
            ).astype(jnp.bfloat16)

        mm_block(0)

        rs_sends = {}
        for g in range(N_BLOCKS):
            if g > 0:
                mm_block(g)
            for c in range(g * cpb, (g + 1) * cpb):
                rdma = pltpu.make_async_remote_copy(
                    src_ref=partial.at[pl.ds(c * mc, mc), :],
                    dst_ref=rs_buf.at[my_pos],
                    send_sem=s1.at[c],
                    recv_sem=r1.at[my_pos],
                    device_id=(c,),
                    device_id_type=pl.DeviceIdType.MESH,
                )
                rs_sends[c] = rdma

                @pl.when(c != my_pos)
                def _(rdma=rdma):
                    rdma.start()

        red_buf[...] = partial[pl.ds(my_pos * mc, mc), :].astype(jnp.float32)

        for q in range(N_DEV):
            @pl.when(q != my_pos)
            def _(q=q):
                recv = pltpu.make_async_remote_copy(
                    src_ref=rs_buf.at[q],
                    dst_ref=rs_buf.at[q],
                    send_sem=s1.at[q],
                    recv_sem=r1.at[q],
                    device_id=(q,),
                    device_id_type=pl.DeviceIdType.MESH,
                )
                recv.wait_recv()
                red_buf[...] += rs_buf[q].astype(jnp.float32)

        out_ref[pl.ds(my_pos * mc, mc), :] = red_buf[...]
        ag_buf[...] = red_buf[...].astype(jnp.bfloat16)

        ag_sends = {}
        for c in range(N_DEV):
            rdma = pltpu.make_async_remote_copy(
                src_ref=ag_buf,
                dst_ref=ag_recv.at[my_pos],
                send_sem=s2.at[c],
                recv_sem=r2.at[my_pos],
                device_id=(c,),
                device_id_type=pl.DeviceIdType.MESH,
            )
            ag_sends[c] = rdma

            @pl.when(c != my_pos)
            def _(rdma=rdma):
                rdma.start()

        for q in range(N_DEV):
            @pl.when(q != my_pos)
            def _(q=q):
                recv = pltpu.make_async_remote_copy(
                    src_ref=ag_recv.at[q],
                    dst_ref=ag_recv.at[q],
                    send_sem=s2.at[q],
                    recv_sem=r2.at[q],
                    device_id=(q,),
                    device_id_type=pl.DeviceIdType.MESH,
                )
                recv.wait_recv()
                out_ref[pl.ds(q * mc, mc), :] = ag_recv[q].astype(jnp.float32)

        for c in range(N_DEV):
            @pl.when(c != my_pos)
            def _(c=c):
                rs_sends[c].wait_send()
                ag_sends[c].wait_send()

    return pl.pallas_call(
        body,
        out_shape=jax.ShapeDtypeStruct((m, n), jnp.float32),
        in_specs=[
            pl.BlockSpec(memory_space=pltpu.VMEM),
            pl.BlockSpec(memory_space=pltpu.VMEM),
        ],
        out_specs=pl.BlockSpec(memory_space=pltpu.VMEM),
        scratch_shapes=[
            pltpu.VMEM((m, n), jnp.bfloat16),
            pltpu.VMEM((N_DEV, mc, n), jnp.bfloat16),
            pltpu.VMEM((mc, n), jnp.float32),
            pltpu.VMEM((mc, n), jnp.bfloat16),
            pltpu.VMEM((N_DEV, mc, n), jnp.bfloat16),
            pltpu.SemaphoreType.DMA((N_DEV,)),
            pltpu.SemaphoreType.DMA((N_DEV,)),
            pltpu.SemaphoreType.DMA((N_DEV,)),
            pltpu.SemaphoreType.DMA((N_DEV,)),
        ],
        compiler_params=pltpu.CompilerParams(collective_id=0),
    )(A, B)
